# baseline (speedup 1.0000x reference)
import functools

import jax
import jax.numpy as jnp
from jax import lax
from jax.experimental import pallas as pl
from jax.experimental.pallas import tpu as pltpu
from jax.experimental.pallas import tpu_sc as plsc

B = 16384
F = 26
E = 16
FIELD_DIM = 100000
TOTAL_ROWS = F * FIELD_DIM

NW = 32
BPW = B // NW
CB = 16
NCHUNK = BPW // CB
ROWS = CB * F
SPC = 4
SLEN = ROWS // SPC
NPAIR = NCHUNK // 2

FIDX = BPW * F // 128


TROWS = 2600000
TCHUNK = 1024
NBLK = TROWS // TCHUNK
CPW = -(-NBLK // NW)
TAIL = TROWS - NBLK * TCHUNK


def _repack_body(embt_hbm, tail_hbm, dep_hbm, in0, in1, out0, out1,
                 tailin, tailb, isem0, isem1, osem0, osem1):
    cid = lax.axis_index("c")
    sid = lax.axis_index("s")
    wid = sid * 2 + cid

    iot = lax.iota(jnp.int32, 16)

    def blk(cl):
        return cl * NW + wid

    def ok(b):
        return b < NBLK

    def fire_in(b, in_v, isem):
        pltpu.async_copy(embt_hbm.at[:, pl.ds(b * TCHUNK, TCHUNK)],
                         in_v, isem)

    sr = (iot * E) >> 7
    sc = [(iot * E + e) & 127 for e in range(E)]

    def transpose(in_v, out_v):
        def blk_body(bb, carry):
            r0 = bb * E
            rows2 = sr + bb * 2
            for e in range(E):
                v = in_v[e, pl.ds(r0, E)]
                plsc.store_scatter(out_v, [rows2, sc[e]], v)
            return carry
        lax.fori_loop(0, TCHUNK // E, blk_body, 0, unroll=4)

    def fire_out(b, out_v, osem):
        pltpu.async_copy(out_v,
                         dep_hbm.at[pl.ds(b * (TCHUNK // 8), TCHUNK // 8)],
                         osem)

    def wait_in(in_v, isem):
        pltpu.make_async_copy(embt_hbm.at[:, pl.ds(0, TCHUNK)], in_v,
                              isem).wait()

    def wait_out(b, out_v, osem):
        pltpu.make_async_copy(out_v,
                              dep_hbm.at[pl.ds(b * (TCHUNK // 8),
                                               TCHUNK // 8)], osem).wait()

    @pl.when(ok(blk(0)))
    def _():
        fire_in(blk(0), in0, isem0)

    def pair_body(cc, carry):
        c0 = 2 * cc
        c1 = c0 + 1
        b0, b1 = blk(c0), blk(c1)

        @pl.when(ok(b1))
        def _():
            fire_in(b1, in1, isem1)

        @pl.when(ok(b0))
        def _():
            @pl.when(cc > 0)
            def _():
                wait_out(blk(c0 - 2), out0, osem0)
            wait_in(in0, isem0)
            transpose(in0, out0)
            fire_out(b0, out0, osem0)

        @pl.when(ok(blk(c0 + 2)))
        def _():
            fire_in(blk(c0 + 2), in0, isem0)

        @pl.when(ok(b1))
        def _():
            @pl.when(cc > 0)
            def _():
                wait_out(blk(c1 - 2), out1, osem1)
            wait_in(in1, isem1)
            transpose(in1, out1)
            fire_out(b1, out1, osem1)
        return carry
    lax.fori_loop(0, CPW // 2, pair_body, 0)

    last0, last1 = blk(CPW - 2), blk(CPW - 1)

    @pl.when(ok(last0))
    def _():
        wait_out(last0, out0, osem0)

    @pl.when(ok(last1))
    def _():
        wait_out(last1, out1, osem1)

    @pl.when(wid == NW - 1)
    def _():
        pltpu.sync_copy(tail_hbm, tailin)

        def r_body(r, carry):
            v = tailin[r]
            tailb[r // 8, pl.ds((r % 8) * E, E)] = v
            return carry
        lax.fori_loop(0, TAIL, r_body, 0)
        pltpu.sync_copy(tailb,
                        dep_hbm.at[pl.ds(NBLK * TCHUNK // 8, TAIL // 8)])


_sc_repack = functools.partial(
    pl.kernel,
    out_type=jax.ShapeDtypeStruct((TROWS // 8, 128), jnp.float32),
    mesh=plsc.VectorSubcoreMesh(core_axis_name="c", subcore_axis_name="s"),
    compiler_params=pltpu.CompilerParams(needs_layout_passes=False,
                                         use_tc_tiling_on_sc=True),
    scratch_types=[
        pltpu.VMEM((E, TCHUNK), jnp.float32),
        pltpu.VMEM((E, TCHUNK), jnp.float32),
        pltpu.VMEM((TCHUNK // 8, 128), jnp.float32),
        pltpu.VMEM((TCHUNK // 8, 128), jnp.float32),
        pltpu.VMEM((TAIL, E), jnp.float32),
        pltpu.VMEM((TAIL // 8, 128), jnp.float32),
        pltpu.SemaphoreType.DMA,
        pltpu.SemaphoreType.DMA,
        pltpu.SemaphoreType.DMA,
        pltpu.SemaphoreType.DMA,
    ],
)(_repack_body)


def _fm_body(tidx_hbm, rem_hbm, embed128_hbm, fm_hbm,
             tidx0, tidx1, rem0, rem1, rows0, rows1, out_v, sem0, sem1):
    cid = lax.axis_index("c")
    sid = lax.axis_index("s")
    wid = sid * 2 + cid

    def stage_fire(c, tidx_v, rem_v, rows, sem):
        pltpu.sync_copy(tidx_hbm.at[wid, pl.ds(c * SPC, SPC)], tidx_v)
        pltpu.sync_copy(rem_hbm.at[wid, c], rem_v)
        for s in range(SPC):
            pltpu.async_copy(embed128_hbm.at[tidx_v.at[s]],
                             rows.at[pl.ds(s * SLEN, SLEN)], sem)

    def drain(rows, sem):
        pltpu.make_async_copy(embed128_hbm.at[pl.ds(0, ROWS)], rows,
                              sem).wait()

    iot = lax.iota(jnp.int32, 16)
    f_stride = iot * F
    e_splats = [jnp.full((16,), e, jnp.int32) for e in range(E)]

    def compute(c, rem_v, rows):
        def field_body(f, carry):
            slots = f_stride + f
            rems = plsc.load_gather(rem_v, [slots])
            acc = list(carry)
            for e in range(E):
                v = plsc.load_gather(rows, [slots, rems + e_splats[e]])
                acc[e] = acc[e] + v
                acc[E + e] = acc[E + e] + v * v
            return tuple(acc)

        zero = jnp.zeros((16,), jnp.float32)
        acc = lax.fori_loop(0, F, field_body, (zero,) * (2 * E))
        fm = acc[0] * acc[0] - acc[E]
        for e in range(1, E):
            fm = fm + acc[e] * acc[e] - acc[E + e]
        out_v[pl.ds(c * CB, CB)] = fm

    stage_fire(0, tidx0, rem0, rows0, sem0)

    def pair_body(cc, carry):
        c0 = 2 * cc
        stage_fire(c0 + 1, tidx1, rem1, rows1, sem1)
        drain(rows0, sem0)
        compute(c0, rem0, rows0)

        @pl.when(cc < NPAIR - 1)
        def _():
            stage_fire(c0 + 2, tidx0, rem0, rows0, sem0)

        drain(rows1, sem1)
        compute(c0 + 1, rem1, rows1)
        return carry
    lax.fori_loop(0, NPAIR, pair_body, 0)

    pltpu.sync_copy(out_v, fm_hbm.at[pl.ds(wid * BPW, BPW)])


_sc_fm = functools.partial(
    pl.kernel,
    out_type=jax.ShapeDtypeStruct((B,), jnp.float32),
    mesh=plsc.VectorSubcoreMesh(core_axis_name="c", subcore_axis_name="s"),
    compiler_params=pltpu.CompilerParams(needs_layout_passes=False,
                                         use_tc_tiling_on_sc=True),
    scratch_types=[
        pltpu.VMEM((SPC, SLEN), jnp.int32),
        pltpu.VMEM((SPC, SLEN), jnp.int32),
        pltpu.VMEM((ROWS,), jnp.int32),
        pltpu.VMEM((ROWS,), jnp.int32),
        pltpu.VMEM((ROWS, 128), jnp.float32),
        pltpu.VMEM((ROWS, 128), jnp.float32),
        pltpu.VMEM((BPW,), jnp.float32),
        pltpu.SemaphoreType.DMA,
        pltpu.SemaphoreType.DMA,
    ],
)(_fm_body)


def _fin_body(idx_hbm, fc_hbm, fm_hbm, bias_hbm, out_hbm,
              idx_v, fc_v, fm_v, out_v, bias_v, sem):
    cid = lax.axis_index("c")
    sid = lax.axis_index("s")
    wid = sid * 2 + cid

    pltpu.sync_copy(idx_hbm.at[wid], idx_v)
    pltpu.sync_copy(fm_hbm.at[pl.ds(wid * BPW, BPW)], fm_v)
    pltpu.sync_copy(bias_hbm, bias_v)
    bias_vec = bias_v[...]

    def fire_body(cc, carry):
        for j in range(13):
            r = cc * 13 + j
            pltpu.async_copy(fc_hbm.at[idx_v.at[r]],
                             fc_v.at[pl.ds(r * 128, 128)], sem)
        return carry
    lax.fori_loop(0, FIDX // 13, fire_body, 0)
    pltpu.make_async_copy(fc_hbm.at[pl.ds(0, BPW * F)], fc_v, sem).wait()

    iot = lax.iota(jnp.int32, 16)
    f_stride = iot * F

    def group_body(g, carry):
        base = g * (16 * F) + f_stride
        fsum = plsc.load_gather(fc_v, [base])
        for f in range(1, F):
            fsum = fsum + plsc.load_gather(fc_v, [base + f])
        fm16 = fm_v[pl.ds(g * 16, 16)]
        y = 0.5 * fm16 + fsum + bias_vec
        out_v[pl.ds(g * 16, 16)] = 1.0 / (1.0 + jnp.exp(-y))
        return carry
    lax.fori_loop(0, BPW // 16, group_body, 0)

    pltpu.sync_copy(out_v, out_hbm.at[pl.ds(wid * BPW, BPW)])


_sc_fin = functools.partial(
    pl.kernel,
    out_type=jax.ShapeDtypeStruct((B,), jnp.float32),
    mesh=plsc.VectorSubcoreMesh(core_axis_name="c", subcore_axis_name="s"),
    compiler_params=pltpu.CompilerParams(needs_layout_passes=False,
                                         use_tc_tiling_on_sc=False),
    scratch_types=[
        pltpu.VMEM((FIDX, 128), jnp.int32),
        pltpu.VMEM((BPW * F,), jnp.float32),
        pltpu.VMEM((BPW,), jnp.float32),
        pltpu.VMEM((BPW,), jnp.float32),
        pltpu.VMEM((16,), jnp.float32),
        pltpu.SemaphoreType.DMA,
    ],
)(_fin_body)


@jax.jit
def kernel(xx, embed_table, fc_table, bias):
    offsets = (jnp.arange(F, dtype=jnp.int32) * FIELD_DIM)[None, :]
    idx = (xx.astype(jnp.int32) + offsets).reshape(NW, BPW * F)
    tidx = (idx >> 3).reshape(NW, NCHUNK * SPC, SLEN)
    rem = ((idx & 7) << 4).reshape(NW, NCHUNK, ROWS)
    embed128 = _sc_repack(embed_table.T, embed_table[TROWS - TAIL:])
    fm = _sc_fm(tidx, rem, embed128)
    idxb = idx.reshape(NW, FIDX, 128)
    fc_flat = fc_table.reshape(TOTAL_ROWS)
    bias16 = jnp.broadcast_to(bias.astype(jnp.float32), (16,))
    return _sc_fin(idxb, fc_flat, fm, bias16)


# speedup vs baseline: 1.3157x; 1.3157x over previous
import functools

import jax
import jax.numpy as jnp
from jax import lax
from jax.experimental import pallas as pl
from jax.experimental.pallas import tpu as pltpu
from jax.experimental.pallas import tpu_sc as plsc

B = 16384
F = 26
E = 16
FIELD_DIM = 100000
TOTAL_ROWS = F * FIELD_DIM

NW = 32
BPW = B // NW
CB = 16
NCHUNK = BPW // CB
ROWS = CB * F
SPC = 4
SLEN = ROWS // SPC
NPAIR = NCHUNK // 2

FIDX = BPW * F // 128


TROWS = 2600000
TCHUNK = 1024
NBLK = TROWS // TCHUNK
CPW = -(-NBLK // NW)
TAIL = TROWS - NBLK * TCHUNK


def _repack_body(embt_hbm, tail_hbm, dep_hbm, in0, in1, out0, out1,
                 tailin, tailb, isem0, isem1, osem0, osem1):
    cid = lax.axis_index("c")
    sid = lax.axis_index("s")
    wid = sid * 2 + cid

    iot = lax.iota(jnp.int32, 16)

    def blk(cl):
        return cl * NW + wid

    def ok(b):
        return b < NBLK

    def fire_in(b, in_v, isem):
        pltpu.async_copy(embt_hbm.at[:, pl.ds(b * TCHUNK, TCHUNK)],
                         in_v, isem)

    sc = [iot * 8 + p for p in range(8)]
    z16 = jnp.zeros((16,), jnp.int32)

    def transpose(in_v, out_v):
        def blk_body(bb, carry):
            r0 = bb * E
            rowv = z16 + bb
            for p in range(8):
                a = in_v[2 * p, pl.ds(r0, E)]
                b = in_v[2 * p + 1, pl.ds(r0, E)]
                pk = plsc.pack(a, b, format=plsc.PackFormat.INTERLEAVED)
                w = plsc.bitcast(pk, jnp.int32)
                plsc.store_scatter(out_v, [rowv, sc[p]], w)
            return carry
        lax.fori_loop(0, TCHUNK // E, blk_body, 0, unroll=4)

    def fire_out(b, out_v, osem):
        pltpu.async_copy(out_v,
                         dep_hbm.at[pl.ds(b * (TCHUNK // 16), TCHUNK // 16)],
                         osem)

    def wait_in(in_v, isem):
        pltpu.make_async_copy(embt_hbm.at[:, pl.ds(0, TCHUNK)], in_v,
                              isem).wait()

    def wait_out(b, out_v, osem):
        pltpu.make_async_copy(out_v,
                              dep_hbm.at[pl.ds(b * (TCHUNK // 16),
                                               TCHUNK // 16)], osem).wait()

    @pl.when(ok(blk(0)))
    def _():
        fire_in(blk(0), in0, isem0)

    def pair_body(cc, carry):
        c0 = 2 * cc
        c1 = c0 + 1
        b0, b1 = blk(c0), blk(c1)

        @pl.when(ok(b1))
        def _():
            fire_in(b1, in1, isem1)

        @pl.when(ok(b0))
        def _():
            @pl.when(cc > 0)
            def _():
                wait_out(blk(c0 - 2), out0, osem0)
            wait_in(in0, isem0)
            transpose(in0, out0)
            fire_out(b0, out0, osem0)

        @pl.when(ok(blk(c0 + 2)))
        def _():
            fire_in(blk(c0 + 2), in0, isem0)

        @pl.when(ok(b1))
        def _():
            @pl.when(cc > 0)
            def _():
                wait_out(blk(c1 - 2), out1, osem1)
            wait_in(in1, isem1)
            transpose(in1, out1)
            fire_out(b1, out1, osem1)
        return carry
    lax.fori_loop(0, CPW // 2, pair_body, 0)

    last0, last1 = blk(CPW - 2), blk(CPW - 1)

    @pl.when(ok(last0))
    def _():
        wait_out(last0, out0, osem0)

    @pl.when(ok(last1))
    def _():
        wait_out(last1, out1, osem1)

    @pl.when(wid == NW - 1)
    def _():
        pltpu.sync_copy(tail_hbm, tailin)
        for bb in range(TAIL // E):
            ridx = bb * E + iot
            rowv = z16 + bb
            for p in range(8):
                a = plsc.load_gather(tailin, [ridx,
                                              jnp.full((16,), 2 * p,
                                                       jnp.int32)])
                b = plsc.load_gather(tailin, [ridx,
                                              jnp.full((16,), 2 * p + 1,
                                                       jnp.int32)])
                pk = plsc.pack(a, b, format=plsc.PackFormat.INTERLEAVED)
                w = plsc.bitcast(pk, jnp.int32)
                plsc.store_scatter(tailb, [rowv, sc[p]], w)
        pltpu.sync_copy(tailb,
                        dep_hbm.at[pl.ds(NBLK * TCHUNK // 16, TAIL // 16)])


_sc_repack = functools.partial(
    pl.kernel,
    out_type=jax.ShapeDtypeStruct((TROWS // 16, 128), jnp.int32),
    mesh=plsc.VectorSubcoreMesh(core_axis_name="c", subcore_axis_name="s"),
    compiler_params=pltpu.CompilerParams(needs_layout_passes=False,
                                         use_tc_tiling_on_sc=True),
    scratch_types=[
        pltpu.VMEM((E, TCHUNK), jnp.float32),
        pltpu.VMEM((E, TCHUNK), jnp.float32),
        pltpu.VMEM((TCHUNK // 16, 128), jnp.int32),
        pltpu.VMEM((TCHUNK // 16, 128), jnp.int32),
        pltpu.VMEM((TAIL, E), jnp.float32),
        pltpu.VMEM((TAIL // 16, 128), jnp.int32),
        pltpu.SemaphoreType.DMA,
        pltpu.SemaphoreType.DMA,
        pltpu.SemaphoreType.DMA,
        pltpu.SemaphoreType.DMA,
    ],
)(_repack_body)


def _fm_body(tidx_hbm, dep8_hbm, fm_hbm,
             tidx0, tidx1, rows0, rows1, out_v, sem0, sem1):
    cid = lax.axis_index("c")
    sid = lax.axis_index("s")
    wid = sid * 2 + cid

    def stage_fire(c, tidx_v, rows, sem):
        pltpu.sync_copy(tidx_hbm.at[wid, pl.ds(c * SPC, SPC)], tidx_v)
        for s in range(SPC):
            pltpu.async_copy(dep8_hbm.at[tidx_v.at[s]],
                             rows.at[pl.ds(s * SLEN, SLEN)], sem)

    def drain(rows, sem):
        pltpu.make_async_copy(dep8_hbm.at[pl.ds(0, ROWS)], rows, sem).wait()

    iot = lax.iota(jnp.int32, 16)
    f_stride = iot * F
    p_splats = [jnp.full((16,), p, jnp.int32) for p in range(8)]

    def compute(c, rows_w):

        def field_body(f, carry):
            slots = f_stride + f
            acc = list(carry)
            for p in range(8):
                w = plsc.load_gather(rows_w, [slots, p_splats[p]])
                pk = plsc.bitcast(w, jnp.bfloat16)
                a, b = plsc.unpack(pk, format=plsc.PackFormat.INTERLEAVED)
                a = a.astype(jnp.float32)
                b = b.astype(jnp.float32)
                acc[2 * p] = acc[2 * p] + a
                acc[E + 2 * p] = acc[E + 2 * p] + a * a
                acc[2 * p + 1] = acc[2 * p + 1] + b
                acc[E + 2 * p + 1] = acc[E + 2 * p + 1] + b * b
            return tuple(acc)

        zero = jnp.zeros((16,), jnp.float32)
        acc = lax.fori_loop(0, F, field_body, (zero,) * (2 * E))
        fm = acc[0] * acc[0] - acc[E]
        for e in range(1, E):
            fm = fm + acc[e] * acc[e] - acc[E + e]
        out_v[pl.ds(c * CB, CB)] = fm

    stage_fire(0, tidx0, rows0, sem0)

    def pair_body(cc, carry):
        c0 = 2 * cc
        stage_fire(c0 + 1, tidx1, rows1, sem1)
        drain(rows0, sem0)
        compute(c0, rows0)

        @pl.when(cc < NPAIR - 1)
        def _():
            stage_fire(c0 + 2, tidx0, rows0, sem0)

        drain(rows1, sem1)
        compute(c0 + 1, rows1)
        return carry
    lax.fori_loop(0, NPAIR, pair_body, 0)

    pltpu.sync_copy(out_v, fm_hbm.at[pl.ds(wid * BPW, BPW)])


_sc_fm = functools.partial(
    pl.kernel,
    out_type=jax.ShapeDtypeStruct((B,), jnp.float32),
    mesh=plsc.VectorSubcoreMesh(core_axis_name="c", subcore_axis_name="s"),
    compiler_params=pltpu.CompilerParams(needs_layout_passes=False,
                                         use_tc_tiling_on_sc=False),
    scratch_types=[
        pltpu.VMEM((SPC, SLEN), jnp.int32),
        pltpu.VMEM((SPC, SLEN), jnp.int32),
        pltpu.VMEM((ROWS, 8), jnp.int32),
        pltpu.VMEM((ROWS, 8), jnp.int32),
        pltpu.VMEM((BPW,), jnp.float32),
        pltpu.SemaphoreType.DMA,
        pltpu.SemaphoreType.DMA,
    ],
)(_fm_body)


def _fin_body(idx_hbm, fc_hbm, fm_hbm, bias_hbm, out_hbm,
              idx_v, fc_v, fm_v, out_v, bias_v, sem):
    cid = lax.axis_index("c")
    sid = lax.axis_index("s")
    wid = sid * 2 + cid

    pltpu.sync_copy(idx_hbm.at[wid], idx_v)
    pltpu.sync_copy(fm_hbm.at[pl.ds(wid * BPW, BPW)], fm_v)
    pltpu.sync_copy(bias_hbm, bias_v)
    bias_vec = bias_v[...]

    def fire_body(cc, carry):
        for j in range(13):
            r = cc * 13 + j
            pltpu.async_copy(fc_hbm.at[idx_v.at[r]],
                             fc_v.at[pl.ds(r * 128, 128)], sem)
        return carry
    lax.fori_loop(0, FIDX // 13, fire_body, 0)
    pltpu.make_async_copy(fc_hbm.at[pl.ds(0, BPW * F)], fc_v, sem).wait()

    iot = lax.iota(jnp.int32, 16)
    f_stride = iot * F

    def group_body(g, carry):
        base = g * (16 * F) + f_stride
        fsum = plsc.load_gather(fc_v, [base])
        for f in range(1, F):
            fsum = fsum + plsc.load_gather(fc_v, [base + f])
        fm16 = fm_v[pl.ds(g * 16, 16)]
        y = 0.5 * fm16 + fsum + bias_vec
        out_v[pl.ds(g * 16, 16)] = 1.0 / (1.0 + jnp.exp(-y))
        return carry
    lax.fori_loop(0, BPW // 16, group_body, 0)

    pltpu.sync_copy(out_v, out_hbm.at[pl.ds(wid * BPW, BPW)])


_sc_fin = functools.partial(
    pl.kernel,
    out_type=jax.ShapeDtypeStruct((B,), jnp.float32),
    mesh=plsc.VectorSubcoreMesh(core_axis_name="c", subcore_axis_name="s"),
    compiler_params=pltpu.CompilerParams(needs_layout_passes=False,
                                         use_tc_tiling_on_sc=False),
    scratch_types=[
        pltpu.VMEM((FIDX, 128), jnp.int32),
        pltpu.VMEM((BPW * F,), jnp.float32),
        pltpu.VMEM((BPW,), jnp.float32),
        pltpu.VMEM((BPW,), jnp.float32),
        pltpu.VMEM((16,), jnp.float32),
        pltpu.SemaphoreType.DMA,
    ],
)(_fin_body)


@jax.jit
def kernel(xx, embed_table, fc_table, bias):
    offsets = (jnp.arange(F, dtype=jnp.int32) * FIELD_DIM)[None, :]
    idx = (xx.astype(jnp.int32) + offsets).reshape(NW, BPW * F)
    tidx = idx.reshape(NW, NCHUNK * SPC, SLEN)
    dep = _sc_repack(embed_table.T, embed_table[TROWS - TAIL:])
    fm = _sc_fm(tidx, dep.reshape(TROWS, 8))
    idxb = idx.reshape(NW, FIDX, 128)
    fc_flat = fc_table.reshape(TOTAL_ROWS)
    bias16 = jnp.broadcast_to(bias.astype(jnp.float32), (16,))
    return _sc_fin(idxb, fc_flat, fm, bias16)
